# Initial kernel scaffold; baseline (speedup 1.0000x reference)
import jax
import jax.numpy as jnp
from jax.experimental import pallas as pl


def kernel(x, edge_index, batch, W1, b1, W2, b2, att_l, att_r):
    raise NotImplementedError("write your pallas kernel here")


# speedup vs baseline: 15.5539x; 15.5539x over previous
import dataclasses
import functools

import jax
import jax.numpy as jnp
from jax import lax
from jax.experimental import pallas as pl
from jax.experimental.pallas import tpu as pltpu
from jax.experimental.pallas import tpu_sc as plsc

N = 10000
E = 320000
D = 128
HID = 128
G = 128
EPS = 0.3

NC = 2
NS = 16
NW = NC * NS
CH = 128
NCHUNK = E // CH
CHUNKS_PER_W = -(-NCHUNK // NW)
NPAD = 10240
STRIPE = NPAD // NS

BLK = 1000
NBLK = N // BLK

_MESH = plsc.VectorSubcoreMesh(
    core_axis_name="c", subcore_axis_name="s", num_cores=NC, num_subcores=NS
)


_SC_PARAMS = pltpu.CompilerParams()
if "needs_layout_passes" in pltpu.CompilerParams.__dataclass_fields__:
    _SC_PARAMS = dataclasses.replace(_SC_PARAMS, needs_layout_passes=False)
_SC_PARAMS_UNTILED = dataclasses.replace(_SC_PARAMS, use_tc_tiling_on_sc=False)


def _f32(*shape):
    return jax.ShapeDtypeStruct(shape, jnp.float32)


def _sc_degree(dst, ones_chunk, zeros_deg):
    @functools.partial(
        pl.kernel,
        out_type=_f32(NC, NPAD, 16),
        mesh=_MESH,
        compiler_params=_SC_PARAMS_UNTILED,
        scratch_types=[
            pltpu.VMEM((CH,), jnp.int32),
            pltpu.VMEM((CH, 16), jnp.float32),
            pltpu.VMEM_SHARED((NPAD, 16), jnp.float32),
        ],
    )
    def k(dst_hbm, ones_hbm, zeros_hbm, degp_hbm, dst_v, ones_v, deg_sh):
        cid = lax.axis_index("c")
        sid = lax.axis_index("s")
        wid = cid * NS + sid
        pltpu.sync_copy(ones_hbm, ones_v)
        pltpu.sync_copy(zeros_hbm, deg_sh.at[pl.ds(sid * STRIPE, STRIPE)])
        plsc.subcore_barrier()

        @pl.loop(0, CHUNKS_PER_W)
        def _(i):
            c = wid + i * NW

            @pl.when(c < NCHUNK)
            def _():
                pltpu.sync_copy(dst_hbm.at[pl.ds(c * CH, CH)], dst_v)
                pltpu.sync_copy(ones_v, deg_sh.at[dst_v], add=True)

        plsc.subcore_barrier()
        pltpu.sync_copy(
            deg_sh.at[pl.ds(sid * STRIPE, STRIPE)],
            degp_hbm.at[cid, pl.ds(sid * STRIPE, STRIPE)],
        )

    return k(dst, ones_chunk, zeros_deg)


def _sc_edge_layer(src, dst, h, al, ar, dis, zeros_rows):
    @functools.partial(
        pl.kernel,
        out_type=_f32(NC, NPAD, HID),
        mesh=_MESH,
        compiler_params=_SC_PARAMS,
        scratch_types=[
            pltpu.VMEM((CH,), jnp.int32),
            pltpu.VMEM((CH,), jnp.int32),
            pltpu.VMEM((CH, HID), jnp.float32),
            pltpu.VMEM((CH,), jnp.float32),
            pltpu.VMEM((NPAD // 128, 128), jnp.float32),
            pltpu.VMEM((NPAD // 128, 128), jnp.float32),
            pltpu.VMEM((NPAD // 128, 128), jnp.float32),
            pltpu.VMEM_SHARED((NPAD, HID), jnp.float32),
        ],
    )
    def k(src_hbm, dst_hbm, h_hbm, al_hbm, ar_hbm, dis_hbm, zeros_hbm, aggp_hbm,
          src_v, dst_v, rows_v, w_v, al_v, ar_v, dis_v, agg_sh):
        cid = lax.axis_index("c")
        sid = lax.axis_index("s")
        wid = cid * NS + sid
        pltpu.sync_copy(al_hbm, al_v)
        pltpu.sync_copy(ar_hbm, ar_v)
        pltpu.sync_copy(dis_hbm, dis_v)
        pltpu.sync_copy(zeros_hbm, agg_sh.at[pl.ds(sid * STRIPE, STRIPE)])
        plsc.subcore_barrier()

        @pl.loop(0, CHUNKS_PER_W)
        def _(i):
            c = wid + i * NW

            @pl.when(c < NCHUNK)
            def _():
                base = c * CH
                pltpu.sync_copy(src_hbm.at[pl.ds(base, CH)], src_v)
                pltpu.sync_copy(dst_hbm.at[pl.ds(base, CH)], dst_v)
                pltpu.sync_copy(h_hbm.at[src_v], rows_v)
                for g in range(CH // 16):
                    s16 = src_v[pl.ds(g * 16, 16)]
                    d16 = dst_v[pl.ds(g * 16, 16)]
                    sr, sc = s16 >> 7, s16 & 127
                    dr, dc = d16 >> 7, d16 & 127
                    al = plsc.load_gather(al_v, [sr, sc])
                    ar = plsc.load_gather(ar_v, [dr, dc])
                    qs = plsc.load_gather(dis_v, [sr, sc])
                    qd = plsc.load_gather(dis_v, [dr, dc])
                    ssum = al + ar
                    e2 = jnp.exp(ssum + ssum)
                    t = 1.0 - 2.0 / (e2 + 1.0)
                    w_v[pl.ds(g * 16, 16)] = t * qs * qd
                for r in range(CH):
                    wb = plsc.load_gather(w_v, [jnp.full((16,), r, jnp.int32)])
                    for f in range(HID // 16):
                        sl = (r, pl.ds(f * 16, 16))
                        rows_v[sl] = rows_v[sl] * wb
                pltpu.sync_copy(rows_v, agg_sh.at[dst_v], add=True)

        plsc.subcore_barrier()
        pltpu.sync_copy(
            agg_sh.at[pl.ds(sid * STRIPE, STRIPE)],
            aggp_hbm.at[cid, pl.ds(sid * STRIPE, STRIPE)],
        )

    return k(src, dst, h, al, ar, dis, zeros_rows)


def _alpha_cols(h, atts_ref):
    al = jnp.sum(h * atts_ref[0:1, :], axis=1, keepdims=True)
    ar = jnp.sum(h * atts_ref[1:2, :], axis=1, keepdims=True)
    return jnp.concatenate([al, ar], axis=1)


def _to_table(col):
    return jnp.pad(col.reshape(N), (0, NPAD - N)).reshape(NPAD // 128, 128)


def _tc_project(x, W1, b1_2d, atts):
    def body(x_ref, w_ref, b_ref, atts_ref, h_ref, aa_ref):
        h = jnp.dot(x_ref[...], w_ref[...], preferred_element_type=jnp.float32)
        h = jnp.maximum(h + b_ref[...], 0.0)
        h_ref[...] = h
        aa_ref[...] = _alpha_cols(h, atts_ref)

    return pl.pallas_call(
        body,
        grid=(NBLK,),
        in_specs=[
            pl.BlockSpec((BLK, D), lambda i: (i, 0)),
            pl.BlockSpec((D, HID), lambda i: (0, 0)),
            pl.BlockSpec((1, HID), lambda i: (0, 0)),
            pl.BlockSpec((2, HID), lambda i: (0, 0)),
        ],
        out_specs=[
            pl.BlockSpec((BLK, HID), lambda i: (i, 0)),
            pl.BlockSpec((BLK, 2), lambda i: (i, 0)),
        ],
        out_shape=[_f32(N, HID), _f32(N, 2)],
    )(x, W1, b1_2d, atts)


def _tc_dis(degp):
    def body(degp_ref, dis_ref):
        deg = degp_ref[0, :, 0:1] + degp_ref[1, :, 0:1]
        dis_ref[...] = jnp.where(deg > 0.0, lax.rsqrt(jnp.maximum(deg, 1.0)), 0.0)

    return pl.pallas_call(
        body,
        grid=(NBLK,),
        in_specs=[pl.BlockSpec((NC, BLK, 16), lambda i: (0, i, 0))],
        out_specs=pl.BlockSpec((BLK, 1), lambda i: (i, 0)),
        out_shape=_f32(N, 1),
    )(degp)


def _tc_combine(aggp, raw, atts):
    def body(aggp_ref, raw_ref, atts_ref, h_ref, aa_ref):
        h = EPS * raw_ref[...] + aggp_ref[0] + aggp_ref[1]
        h_ref[...] = h
        aa_ref[...] = _alpha_cols(h, atts_ref)

    return pl.pallas_call(
        body,
        grid=(NBLK,),
        in_specs=[
            pl.BlockSpec((NC, BLK, HID), lambda i: (0, i, 0)),
            pl.BlockSpec((BLK, HID), lambda i: (i, 0)),
            pl.BlockSpec((2, HID), lambda i: (0, 0)),
        ],
        out_specs=[
            pl.BlockSpec((BLK, HID), lambda i: (i, 0)),
            pl.BlockSpec((BLK, 2), lambda i: (i, 0)),
        ],
        out_shape=[_f32(N, HID), _f32(N, 2)],
    )(aggp, raw, atts)


def _tc_final(aggp, raw, W2, b2_2d, batch_2d):
    def body(aggp_ref, raw_ref, w_ref, b_ref, batch_ref, out_ref):
        i = pl.program_id(0)
        h2 = EPS * raw_ref[...] + aggp_ref[0] + aggp_ref[1]
        o = jnp.dot(h2, w_ref[...], preferred_element_type=jnp.float32)
        o = o + b_ref[...]
        m = (lax.broadcasted_iota(jnp.int32, (G, BLK), 0) == batch_ref[0])
        p = jnp.dot(m.astype(jnp.float32), o, preferred_element_type=jnp.float32)

        @pl.when(i == 0)
        def _():
            out_ref[...] = jnp.zeros_like(out_ref)

        out_ref[...] += p

    return pl.pallas_call(
        body,
        grid=(NBLK,),
        in_specs=[
            pl.BlockSpec((NC, BLK, HID), lambda i: (0, i, 0)),
            pl.BlockSpec((BLK, HID), lambda i: (i, 0)),
            pl.BlockSpec((HID, HID), lambda i: (0, 0)),
            pl.BlockSpec((1, HID), lambda i: (0, 0)),
            pl.BlockSpec((1, 1, BLK), lambda i: (i, 0, 0)),
        ],
        out_specs=pl.BlockSpec((G, HID), lambda i: (0, 0)),
        out_shape=_f32(G, HID),
    )(aggp, raw, W2, b2_2d, batch_2d)


def kernel(x, edge_index, batch, W1, b1, W2, b2, att_l, att_r):
    src = edge_index[0]
    dst = edge_index[1]
    b1_2d = b1.reshape(1, HID)
    b2_2d = b2.reshape(1, HID)
    batch_2d = batch.reshape(NBLK, 1, BLK)
    atts0 = jnp.stack([att_l[0], att_r[0]])
    atts1 = jnp.stack([att_l[1], att_r[1]])

    ones_chunk = jnp.ones((CH, 16), jnp.float32)
    zeros_deg = jnp.zeros((STRIPE, 16), jnp.float32)
    zeros_rows = jnp.zeros((STRIPE, HID), jnp.float32)

    h0, aa0 = _tc_project(x, W1, b1_2d, atts0)
    degp = _sc_degree(dst, ones_chunk, zeros_deg)
    dis = _to_table(_tc_dis(degp))

    al0, ar0 = _to_table(aa0[:, 0:1]), _to_table(aa0[:, 1:2])
    aggp0 = _sc_edge_layer(src, dst, h0, al0, ar0, dis, zeros_rows)
    h1, aa1 = _tc_combine(aggp0, h0, atts1)
    al1, ar1 = _to_table(aa1[:, 0:1]), _to_table(aa1[:, 1:2])
    aggp1 = _sc_edge_layer(src, dst, h1, al1, ar1, dis, zeros_rows)
    return _tc_final(aggp1, h0, W2, b2_2d, batch_2d)


# speedup vs baseline: 25.3592x; 1.6304x over previous
import dataclasses
import functools

import jax
import jax.numpy as jnp
from jax import lax
from jax.experimental import pallas as pl
from jax.experimental.pallas import tpu as pltpu
from jax.experimental.pallas import tpu_sc as plsc

N = 10000
E = 320000
D = 128
HID = 128
G = 128
EPS = 0.3

NC = 2
NS = 16
NW = NC * NS
CH = 128
NCHUNK = E // CH
CHUNKS_PER_W = -(-NCHUNK // NW)
NPAD = 10240
STRIPE = NPAD // NS
CHE = 64
CPW = 160
EPAD = CHE * CPW * NW
NBUF = 3

BLK = 1000
NBLK = N // BLK

_MESH = plsc.VectorSubcoreMesh(
    core_axis_name="c", subcore_axis_name="s", num_cores=NC, num_subcores=NS
)

_SC_PARAMS = pltpu.CompilerParams()
if "needs_layout_passes" in pltpu.CompilerParams.__dataclass_fields__:
    _SC_PARAMS = dataclasses.replace(_SC_PARAMS, needs_layout_passes=False)
_SC_PARAMS_UNTILED = dataclasses.replace(_SC_PARAMS, use_tc_tiling_on_sc=False)


def _f32(*shape):
    return jax.ShapeDtypeStruct(shape, jnp.float32)


def _sc_degree(dst, ones_chunk, zeros_deg):
    @functools.partial(
        pl.kernel,
        out_type=_f32(NC, NPAD, 16),
        mesh=_MESH,
        compiler_params=_SC_PARAMS_UNTILED,
        scratch_types=[
            pltpu.VMEM((CH,), jnp.int32),
            pltpu.VMEM((CH, 16), jnp.float32),
            pltpu.VMEM_SHARED((NPAD, 16), jnp.float32),
        ],
    )
    def k(dst_hbm, ones_hbm, zeros_hbm, degp_hbm, dst_v, ones_v, deg_sh):
        cid = lax.axis_index("c")
        sid = lax.axis_index("s")
        wid = cid * NS + sid
        pltpu.sync_copy(ones_hbm, ones_v)
        pltpu.sync_copy(zeros_hbm, deg_sh.at[pl.ds(sid * STRIPE, STRIPE)])
        plsc.subcore_barrier()

        @pl.loop(0, CHUNKS_PER_W)
        def _(i):
            c = wid + i * NW

            @pl.when(c < NCHUNK)
            def _():
                pltpu.sync_copy(dst_hbm.at[pl.ds(c * CH, CH)], dst_v)
                pltpu.sync_copy(ones_v, deg_sh.at[dst_v], add=True)

        plsc.subcore_barrier()
        pltpu.sync_copy(
            deg_sh.at[pl.ds(sid * STRIPE, STRIPE)],
            degp_hbm.at[cid, pl.ds(sid * STRIPE, STRIPE)],
        )

    return k(dst, ones_chunk, zeros_deg)


def _sc_edge_layer(srcP, dstP, hs, al, ar, zeros_rows):
    @functools.partial(
        pl.kernel,
        out_type=_f32(NC, NPAD, HID),
        mesh=_MESH,
        compiler_params=_SC_PARAMS_UNTILED,
        scratch_types=[
            pltpu.VMEM((NBUF, CHE), jnp.int32),
            pltpu.VMEM((NBUF, CHE), jnp.int32),
            pltpu.VMEM((NBUF, CHE, HID), jnp.float32),
            pltpu.VMEM((CHE,), jnp.float32),
            pltpu.VMEM((NPAD,), jnp.float32),
            pltpu.VMEM((NPAD,), jnp.float32),
            pltpu.VMEM_SHARED((NPAD, HID), jnp.float32),
            pltpu.SemaphoreType.DMA((NBUF,)),
            pltpu.SemaphoreType.DMA((NBUF,)),
        ],
    )
    def k(src_hbm, dst_hbm, hs_hbm, al_hbm, ar_hbm, zeros_hbm, aggp_hbm,
          src_v, dst_v, rows_v, w_v, al_v, ar_v, agg_sh, gsem, ssem):
        cid = lax.axis_index("c")
        sid = lax.axis_index("s")
        wid = cid * NS + sid
        base = wid * CPW
        pltpu.sync_copy(al_hbm, al_v)
        pltpu.sync_copy(ar_hbm, ar_v)
        pltpu.sync_copy(zeros_hbm, agg_sh.at[pl.ds(sid * STRIPE, STRIPE)])
        plsc.subcore_barrier()

        def load_ids(c, b):
            pltpu.sync_copy(src_hbm.at[pl.ds((base + c) * CHE, CHE)],
                            src_v.at[b])
            pltpu.sync_copy(dst_hbm.at[pl.ds((base + c) * CHE, CHE)],
                            dst_v.at[b])

        def start_gather(b):
            pltpu.async_copy(hs_hbm.at[src_v.at[b]], rows_v.at[b], gsem.at[b])

        def wait_gather(b):
            pltpu.make_async_copy(
                hs_hbm.at[src_v.at[b]], rows_v.at[b], gsem.at[b]
            ).wait()

        def start_scatter(b):
            pltpu.async_copy(rows_v.at[b], agg_sh.at[dst_v.at[b]], ssem.at[b],
                             add=True)

        def wait_scatter(b):
            pltpu.make_async_copy(
                rows_v.at[b], agg_sh.at[dst_v.at[b]], ssem.at[b]
            ).wait()

        load_ids(0, 0)
        start_gather(0)

        @pl.loop(0, CPW)
        def _(i):
            b = lax.rem(i, NBUF)
            bn = lax.rem(i + 1, NBUF)

            @pl.when(i + 1 < CPW)
            def _():
                @pl.when(i >= NBUF - 1)
                def _():
                    wait_scatter(bn)

                load_ids(i + 1, bn)
                start_gather(bn)

            wait_gather(b)
            for g in range(CHE // 16):
                s16 = src_v[b, pl.ds(g * 16, 16)]
                d16 = dst_v[b, pl.ds(g * 16, 16)]
                alv = plsc.load_gather(al_v, [s16])
                arv = plsc.load_gather(ar_v, [d16])
                ssum = alv + arv
                e2 = jnp.exp(ssum + ssum)
                w_v[pl.ds(g * 16, 16)] = 1.0 - 2.0 / (e2 + 1.0)
            for r in range(CHE):
                wb = plsc.load_gather(w_v, [jnp.full((16,), r, jnp.int32)])
                for f in range(HID // 16):
                    sl = (b, r, pl.ds(f * 16, 16))
                    rows_v[sl] = rows_v[sl] * wb
            start_scatter(b)

        for kk in range(1, NBUF + 1):
            wait_scatter((CPW - kk) % NBUF)

        plsc.subcore_barrier()
        pltpu.sync_copy(
            agg_sh.at[pl.ds(sid * STRIPE, STRIPE)],
            aggp_hbm.at[cid, pl.ds(sid * STRIPE, STRIPE)],
        )

    return k(srcP, dstP, hs, al, ar, zeros_rows)


def _alpha_cols(h, atts_ref):
    al = jnp.sum(h * atts_ref[0:1, :], axis=1, keepdims=True)
    ar = jnp.sum(h * atts_ref[1:2, :], axis=1, keepdims=True)
    return jnp.concatenate([al, ar], axis=1)


def _to_table(col):
    return jnp.pad(col.reshape(N), (0, NPAD - N))


def _tc_project(x, W1, b1_2d, atts):
    def body(x_ref, w_ref, b_ref, atts_ref, h_ref, aa_ref):
        h = jnp.dot(x_ref[...], w_ref[...], preferred_element_type=jnp.float32)
        h = jnp.maximum(h + b_ref[...], 0.0)
        h_ref[...] = h
        aa_ref[...] = _alpha_cols(h, atts_ref)

    return pl.pallas_call(
        body,
        grid=(NBLK,),
        in_specs=[
            pl.BlockSpec((BLK, D), lambda i: (i, 0)),
            pl.BlockSpec((D, HID), lambda i: (0, 0)),
            pl.BlockSpec((1, HID), lambda i: (0, 0)),
            pl.BlockSpec((2, HID), lambda i: (0, 0)),
        ],
        out_specs=[
            pl.BlockSpec((BLK, HID), lambda i: (i, 0)),
            pl.BlockSpec((BLK, 2), lambda i: (i, 0)),
        ],
        out_shape=[_f32(N, HID), _f32(N, 2)],
    )(x, W1, b1_2d, atts)


def _tc_dis_scale(degp, h0):
    def body(degp_ref, h_ref, dis_ref, hs_ref):
        deg = degp_ref[0, :, 0:1] + degp_ref[1, :, 0:1]
        dis = jnp.where(deg > 0.0, lax.rsqrt(jnp.maximum(deg, 1.0)), 0.0)
        dis_ref[...] = dis
        hs_ref[...] = h_ref[...] * dis

    return pl.pallas_call(
        body,
        grid=(NBLK,),
        in_specs=[
            pl.BlockSpec((NC, BLK, 16), lambda i: (0, i, 0)),
            pl.BlockSpec((BLK, HID), lambda i: (i, 0)),
        ],
        out_specs=[
            pl.BlockSpec((BLK, 1), lambda i: (i, 0)),
            pl.BlockSpec((BLK, HID), lambda i: (i, 0)),
        ],
        out_shape=[_f32(N, 1), _f32(N, HID)],
    )(degp, h0)


def _tc_combine(aggp, raw, dis_col, atts):
    def body(aggp_ref, raw_ref, dis_ref, atts_ref, hs_ref, aa_ref):
        dis = dis_ref[...]
        h = EPS * raw_ref[...] + dis * (aggp_ref[0] + aggp_ref[1])
        aa_ref[...] = _alpha_cols(h, atts_ref)
        hs_ref[...] = h * dis

    return pl.pallas_call(
        body,
        grid=(NBLK,),
        in_specs=[
            pl.BlockSpec((NC, BLK, HID), lambda i: (0, i, 0)),
            pl.BlockSpec((BLK, HID), lambda i: (i, 0)),
            pl.BlockSpec((BLK, 1), lambda i: (i, 0)),
            pl.BlockSpec((2, HID), lambda i: (0, 0)),
        ],
        out_specs=[
            pl.BlockSpec((BLK, HID), lambda i: (i, 0)),
            pl.BlockSpec((BLK, 2), lambda i: (i, 0)),
        ],
        out_shape=[_f32(N, HID), _f32(N, 2)],
    )(aggp, raw, dis_col, atts)


def _tc_final(aggp, raw, dis_col, W2, b2_2d, batch_2d):
    def body(aggp_ref, raw_ref, dis_ref, w_ref, b_ref, batch_ref, out_ref):
        i = pl.program_id(0)
        h2 = EPS * raw_ref[...] + dis_ref[...] * (aggp_ref[0] + aggp_ref[1])
        o = jnp.dot(h2, w_ref[...], preferred_element_type=jnp.float32)
        o = o + b_ref[...]
        m = (lax.broadcasted_iota(jnp.int32, (G, BLK), 0) == batch_ref[0])
        p = jnp.dot(m.astype(jnp.float32), o, preferred_element_type=jnp.float32)

        @pl.when(i == 0)
        def _():
            out_ref[...] = jnp.zeros_like(out_ref)

        out_ref[...] += p

    return pl.pallas_call(
        body,
        grid=(NBLK,),
        in_specs=[
            pl.BlockSpec((NC, BLK, HID), lambda i: (0, i, 0)),
            pl.BlockSpec((BLK, HID), lambda i: (i, 0)),
            pl.BlockSpec((BLK, 1), lambda i: (i, 0)),
            pl.BlockSpec((HID, HID), lambda i: (0, 0)),
            pl.BlockSpec((1, HID), lambda i: (0, 0)),
            pl.BlockSpec((1, 1, BLK), lambda i: (i, 0, 0)),
        ],
        out_specs=pl.BlockSpec((G, HID), lambda i: (0, 0)),
        out_shape=_f32(G, HID),
    )(aggp, raw, dis_col, W2, b2_2d, batch_2d)


def kernel(x, edge_index, batch, W1, b1, W2, b2, att_l, att_r):
    src = edge_index[0]
    dst = edge_index[1]
    npad_e = EPAD - E
    pad_src = (jnp.arange(npad_e, dtype=jnp.int32) * 67) % N
    pad_dst = N + (jnp.arange(npad_e, dtype=jnp.int32) % (NPAD - N))
    srcP = jnp.concatenate([src, pad_src])
    dstP = jnp.concatenate([dst, pad_dst])
    b1_2d = b1.reshape(1, HID)
    b2_2d = b2.reshape(1, HID)
    batch_2d = batch.reshape(NBLK, 1, BLK)
    atts0 = jnp.stack([att_l[0], att_r[0]])
    atts1 = jnp.stack([att_l[1], att_r[1]])

    ones_chunk = jnp.ones((CH, 16), jnp.float32)
    zeros_deg = jnp.zeros((STRIPE, 16), jnp.float32)
    zeros_rows = jnp.zeros((STRIPE, HID), jnp.float32)

    h0, aa0 = _tc_project(x, W1, b1_2d, atts0)
    degp = _sc_degree(dst, ones_chunk, zeros_deg)
    dis_col, hs0 = _tc_dis_scale(degp, h0)

    al0, ar0 = _to_table(aa0[:, 0:1]), _to_table(aa0[:, 1:2])
    aggp0 = _sc_edge_layer(srcP, dstP, hs0, al0, ar0, zeros_rows)
    hs1, aa1 = _tc_combine(aggp0, h0, dis_col, atts1)
    al1, ar1 = _to_table(aa1[:, 0:1]), _to_table(aa1[:, 1:2])
    aggp1 = _sc_edge_layer(srcP, dstP, hs1, al1, ar1, zeros_rows)
    return _tc_final(aggp1, h0, dis_col, W2, b2_2d, batch_2d)
